# baseline (speedup 1.0000x reference)
import functools

import jax
import jax.numpy as jnp
from jax import lax
from jax.experimental import pallas as pl
from jax.experimental.pallas import tpu as pltpu
from jax.experimental.pallas import tpu_sc as plsc

N_NODES = 10000
N_EDGES = 320000
D = 128

NC = 2
NS = 16
NW = NC * NS
EPT = N_EDGES // NW
CH = 128
G = 16
NCHUNK = -(-EPT // CH)
NCHUNK = ((NCHUNK + G - 1) // G) * G
NGRP = NCHUNK // G
PAD_EPT = NCHUNK * CH
ACC_ROWS = 10240
DUMP_ROW = N_NODES
ZCH = ACC_ROWS // NS // CH
ROWS_PER_TILE_OUT = ACC_ROWS // NS

@functools.cache
def _mesh():
    return plsc.VectorSubcoreMesh(
        core_axis_name="c", subcore_axis_name="s",
        num_cores=NC, num_subcores=NS)


def _const_vmem_rows(ref, nrows, value):
    v16 = jnp.full((16,), value, jnp.float32)

    def body(i, _):
        for l in range(8):
            ref[i, pl.ds(l * 16, 16)] = v16
        return 0

    lax.fori_loop(0, nrows, body, 0)


def _sc_degree_body(dstp_hbm, degp_hbm, dst_v, ones_v, acc, sem):
    cid = lax.axis_index("c")
    sid = lax.axis_index("s")
    wid = sid * NC + cid
    pltpu.sync_copy(dstp_hbm.at[wid], dst_v)
    _const_vmem_rows(ones_v, CH, 0.0)

    def zero_chunk(i, _):
        pltpu.sync_copy(ones_v, acc.at[pl.ds(sid * (ZCH * CH) + i * CH, CH)])
        return 0

    lax.fori_loop(0, ZCH, zero_chunk, 0)
    _const_vmem_rows(ones_v, CH, 1.0)
    plsc.subcore_barrier()

    def edge_chunk(j, _):
        pltpu.sync_copy(ones_v, acc.at[dst_v.at[j]], add=True)
        return 0

    lax.fori_loop(0, NCHUNK, edge_chunk, 0)
    plsc.subcore_barrier()
    pltpu.sync_copy(
        acc.at[pl.ds(sid * ROWS_PER_TILE_OUT, ROWS_PER_TILE_OUT)],
        degp_hbm.at[cid, pl.ds(sid * ROWS_PER_TILE_OUT, ROWS_PER_TILE_OUT)],
    )


@functools.cache
def _sc_degree():
    return pl.kernel(
        _sc_degree_body,
        out_type=jax.ShapeDtypeStruct((NC, ACC_ROWS, D), jnp.float32),
        mesh=_mesh(),
        scratch_types=[
            pltpu.VMEM((NCHUNK, CH), jnp.int32),
            pltpu.VMEM((CH, D), jnp.float32),
            pltpu.VMEM_SHARED((ACC_ROWS, D), jnp.float32),
            pltpu.SemaphoreType.DMA,
        ],
    )


def _sc_aggregate_body(g_hbm, srcp_hbm, dstp_hbm, part_hbm,
                       src_v, dst_v, rows_v, acc, sem):
    cid = lax.axis_index("c")
    sid = lax.axis_index("s")
    wid = sid * NC + cid
    pltpu.sync_copy(srcp_hbm.at[wid], src_v)
    pltpu.sync_copy(dstp_hbm.at[wid], dst_v)
    _const_vmem_rows(rows_v, CH, 0.0)

    def zero_chunk(i, _):
        pltpu.sync_copy(rows_v, acc.at[pl.ds(sid * (ZCH * CH) + i * CH, CH)])
        return 0

    lax.fori_loop(0, ZCH, zero_chunk, 0)
    plsc.subcore_barrier()

    def edge_chunk(j, _):
        pltpu.async_copy(g_hbm.at[src_v.at[j]], rows_v, sem).wait()
        pltpu.sync_copy(rows_v, acc.at[dst_v.at[j]], add=True)
        return 0

    lax.fori_loop(0, NCHUNK, edge_chunk, 0)
    plsc.subcore_barrier()
    pltpu.sync_copy(
        acc.at[pl.ds(sid * ROWS_PER_TILE_OUT, ROWS_PER_TILE_OUT)],
        part_hbm.at[cid, pl.ds(sid * ROWS_PER_TILE_OUT, ROWS_PER_TILE_OUT)],
    )


@functools.cache
def _sc_aggregate():
    return pl.kernel(
        _sc_aggregate_body,
        out_type=jax.ShapeDtypeStruct((NC, ACC_ROWS, D), jnp.float32),
        mesh=_mesh(),
        scratch_types=[
            pltpu.VMEM((NCHUNK, CH), jnp.int32),
            pltpu.VMEM((NCHUNK, CH), jnp.int32),
            pltpu.VMEM((CH, D), jnp.float32),
            pltpu.VMEM_SHARED((ACC_ROWS, D), jnp.float32),
            pltpu.SemaphoreType.DMA,
        ],
    )


RB = 1000


def _dinv_from_degp(degp_blk):
    deg = 1.0 + degp_blk[0, :, 0:1] + degp_blk[1, :, 0:1]
    return lax.rsqrt(deg)


def _tc1_body(x_ref, w_ref, degp_ref, g_ref):
    dinv = _dinv_from_degp(degp_ref[...])
    h = jnp.dot(x_ref[...], w_ref[...], preferred_element_type=jnp.float32)
    g_ref[...] = h * dinv


def _tc2_body(p_ref, g_ref, degp_ref, w_ref, out_ref):
    dinv = _dinv_from_degp(degp_ref[...])
    s = (p_ref[0] + p_ref[1] + g_ref[...]) * dinv
    a = jnp.where(s >= 0, s, 0.3 * s)
    h = jnp.dot(a, w_ref[...], preferred_element_type=jnp.float32)
    out_ref[...] = h * dinv


def _tc3_body(p_ref, g_ref, degp_ref, out_ref):
    dinv = _dinv_from_degp(degp_ref[...])
    s = (p_ref[0] + p_ref[1] + g_ref[...]) * dinv
    out_ref[...] = jnp.maximum(s, 0.0)


_row_spec = pl.BlockSpec((RB, D), lambda i: (i, 0))
_part_spec = pl.BlockSpec((NC, RB, D), lambda i: (0, i, 0))
_degp_spec = pl.BlockSpec((NC, RB, D), lambda i: (0, i, 0))
_w_spec = pl.BlockSpec((D, D), lambda i: (0, 0))
_out_struct = jax.ShapeDtypeStruct((N_NODES, D), jnp.float32)

_tc1 = pl.pallas_call(
    _tc1_body, grid=(N_NODES // RB,),
    in_specs=[_row_spec, _w_spec, _degp_spec],
    out_specs=_row_spec, out_shape=_out_struct,
)
_tc2 = pl.pallas_call(
    _tc2_body, grid=(N_NODES // RB,),
    in_specs=[_part_spec, _row_spec, _degp_spec, _w_spec],
    out_specs=_row_spec, out_shape=_out_struct,
)
_tc3 = pl.pallas_call(
    _tc3_body, grid=(N_NODES // RB,),
    in_specs=[_part_spec, _row_spec, _degp_spec],
    out_specs=_row_spec, out_shape=_out_struct,
)


def kernel(x, edge_index, W1, W2):
    src = edge_index[0].astype(jnp.int32)
    dst = edge_index[1].astype(jnp.int32)
    pad = PAD_EPT - EPT
    srcp = jnp.pad(src.reshape(NW, EPT), ((0, 0), (0, pad))).reshape(
        NW, NCHUNK, CH)
    dstp = jnp.pad(dst.reshape(NW, EPT), ((0, 0), (0, pad)),
                   constant_values=DUMP_ROW).reshape(NW, NCHUNK, CH)

    degp = _sc_degree()(dstp)
    g1 = _tc1(x, W1, degp)
    p1 = _sc_aggregate()(g1, srcp, dstp)
    g2 = _tc2(p1, g1, degp, W2)
    p2 = _sc_aggregate()(g2, srcp, dstp)
    return _tc3(p2, g2, degp)


# speedup vs baseline: 1.0025x; 1.0025x over previous
import functools

import jax
import jax.numpy as jnp
from jax import lax
from jax.experimental import pallas as pl
from jax.experimental.pallas import tpu as pltpu
from jax.experimental.pallas import tpu_sc as plsc

N_NODES = 10000
N_EDGES = 320000
D = 128

NC = 2
NS = 16
NW = NC * NS
EPT = N_EDGES // NW
CH = 128
G = 16
NCHUNK = -(-EPT // CH)
NCHUNK = ((NCHUNK + G - 1) // G) * G
NGRP = NCHUNK // G
PAD_EPT = NCHUNK * CH
ACC_ROWS = 10240
DUMP_ROW = N_NODES
ZCH = ACC_ROWS // NS // CH
ROWS_PER_TILE_OUT = ACC_ROWS // NS

@functools.cache
def _mesh():
    return plsc.VectorSubcoreMesh(
        core_axis_name="c", subcore_axis_name="s",
        num_cores=NC, num_subcores=NS)


def _const_vmem_rows(ref, nrows, value):
    v16 = jnp.full((16,), value, jnp.float32)

    def body(i, _):
        for l in range(8):
            ref[i, pl.ds(l * 16, 16)] = v16
        return 0

    lax.fori_loop(0, nrows, body, 0)


def _sc_degree_body(dstp_hbm, degp_hbm, dst_v, ones_v, acc, sem):
    cid = lax.axis_index("c")
    sid = lax.axis_index("s")
    wid = sid * NC + cid
    pltpu.sync_copy(dstp_hbm.at[wid], dst_v)
    _const_vmem_rows(ones_v, CH, 0.0)

    def zero_chunk(i, _):
        pltpu.sync_copy(ones_v, acc.at[pl.ds(sid * (ZCH * CH) + i * CH, CH)])
        return 0

    lax.fori_loop(0, ZCH, zero_chunk, 0)
    _const_vmem_rows(ones_v, CH, 1.0)
    plsc.subcore_barrier()

    def edge_chunk(j, _):
        pltpu.sync_copy(ones_v, acc.at[dst_v.at[j]], add=True)
        return 0

    lax.fori_loop(0, NCHUNK, edge_chunk, 0)
    plsc.subcore_barrier()
    pltpu.sync_copy(
        acc.at[pl.ds(sid * ROWS_PER_TILE_OUT, ROWS_PER_TILE_OUT)],
        degp_hbm.at[cid, pl.ds(sid * ROWS_PER_TILE_OUT, ROWS_PER_TILE_OUT)],
    )


@functools.cache
def _sc_degree():
    return pl.kernel(
        _sc_degree_body,
        out_type=jax.ShapeDtypeStruct((NC, ACC_ROWS, D), jnp.float32),
        mesh=_mesh(),
        scratch_types=[
            pltpu.VMEM((NCHUNK, CH), jnp.int32),
            pltpu.VMEM((CH, D), jnp.float32),
            pltpu.VMEM_SHARED((ACC_ROWS, D), jnp.float32),
            pltpu.SemaphoreType.DMA,
        ],
    )


def _sc_aggregate_body(g_hbm, srcp_hbm, dstp_hbm, part_hbm,
                       src_v, dst_v, rows_v, acc, sem):
    cid = lax.axis_index("c")
    sid = lax.axis_index("s")
    wid = sid * NC + cid
    pltpu.sync_copy(srcp_hbm.at[wid], src_v)
    pltpu.sync_copy(dstp_hbm.at[wid], dst_v)
    _const_vmem_rows(rows_v, CH, 0.0)

    def zero_chunk(i, _):
        pltpu.sync_copy(rows_v, acc.at[pl.ds(sid * (ZCH * CH) + i * CH, CH)])
        return 0

    lax.fori_loop(0, ZCH, zero_chunk, 0)
    plsc.subcore_barrier()

    def edge_chunk(j, _):
        pltpu.async_copy(g_hbm.at[src_v.at[j]], rows_v, sem).wait()
        pltpu.sync_copy(rows_v, acc.at[dst_v.at[j]], add=True)
        return 0

    lax.fori_loop(0, NCHUNK, edge_chunk, 0)
    plsc.subcore_barrier()
    pltpu.sync_copy(
        acc.at[pl.ds(sid * ROWS_PER_TILE_OUT, ROWS_PER_TILE_OUT)],
        part_hbm.at[cid, pl.ds(sid * ROWS_PER_TILE_OUT, ROWS_PER_TILE_OUT)],
    )


@functools.cache
def _sc_aggregate():
    return pl.kernel(
        _sc_aggregate_body,
        out_type=jax.ShapeDtypeStruct((NC, ACC_ROWS, D), jnp.float32),
        mesh=_mesh(),
        scratch_types=[
            pltpu.VMEM((NCHUNK, CH), jnp.int32),
            pltpu.VMEM((NCHUNK, CH), jnp.int32),
            pltpu.VMEM((CH, D), jnp.float32),
            pltpu.VMEM_SHARED((ACC_ROWS, D), jnp.float32),
            pltpu.SemaphoreType.DMA,
        ],
    )


RB = 1000


def _dinv_from_degp(degp_blk):
    deg = 1.0 + degp_blk[0, :, 0:1] + degp_blk[1, :, 0:1]
    return lax.rsqrt(deg)


def _tc1_body(x_ref, w_ref, degp_ref, g_ref):
    dinv = _dinv_from_degp(degp_ref[...])
    h = jnp.dot(x_ref[...], w_ref[...], preferred_element_type=jnp.float32)
    g_ref[...] = h * dinv


def _tc2_body(p_ref, g_ref, degp_ref, w_ref, out_ref):
    dinv = _dinv_from_degp(degp_ref[...])
    s = (p_ref[0] + p_ref[1] + g_ref[...]) * dinv
    a = jnp.where(s >= 0, s, 0.3 * s)
    h = jnp.dot(a, w_ref[...], preferred_element_type=jnp.float32)
    out_ref[...] = h * dinv


def _tc3_body(p_ref, g_ref, degp_ref, out_ref):
    dinv = _dinv_from_degp(degp_ref[...])
    s = (p_ref[0] + p_ref[1] + g_ref[...]) * dinv
    out_ref[...] = jnp.maximum(s, 0.0)


_row_spec = pl.BlockSpec((RB, D), lambda i: (i, 0))
_part_spec = pl.BlockSpec((NC, RB, D), lambda i: (0, i, 0))
_degp_spec = pl.BlockSpec((NC, RB, D), lambda i: (0, i, 0))
_w_spec = pl.BlockSpec((D, D), lambda i: (0, 0))
_out_struct = jax.ShapeDtypeStruct((N_NODES, D), jnp.float32)

_tc1 = pl.pallas_call(
    _tc1_body, grid=(N_NODES // RB,),
    in_specs=[_row_spec, _w_spec, _degp_spec],
    out_specs=_row_spec, out_shape=_out_struct,
)
_tc2 = pl.pallas_call(
    _tc2_body, grid=(N_NODES // RB,),
    in_specs=[_part_spec, _row_spec, _degp_spec, _w_spec],
    out_specs=_row_spec, out_shape=_out_struct,
)
_tc3 = pl.pallas_call(
    _tc3_body, grid=(N_NODES // RB,),
    in_specs=[_part_spec, _row_spec, _degp_spec],
    out_specs=_row_spec, out_shape=_out_struct,
)


def kernel(x, edge_index, W1, W2):
    src = edge_index[0].astype(jnp.int32)
    dst = edge_index[1].astype(jnp.int32)
    pad = PAD_EPT - EPT
    srcp = jnp.pad(src.reshape(NW, EPT), ((0, 0), (0, pad))).reshape(
        NW, NCHUNK, CH)
    dump = DUMP_ROW + (jnp.arange(pad, dtype=jnp.int32)
                       % (ACC_ROWS - N_NODES))
    dstp = jnp.concatenate(
        [dst.reshape(NW, EPT), jnp.broadcast_to(dump, (NW, pad))],
        axis=1).reshape(NW, NCHUNK, CH)

    degp = _sc_degree()(dstp)
    g1 = _tc1(x, W1, degp)
    p1 = _sc_aggregate()(g1, srcp, dstp)
    g2 = _tc2(p1, g1, degp, W2)
    p2 = _sc_aggregate()(g2, srcp, dstp)
    return _tc3(p2, g2, degp)


# speedup vs baseline: 1.4235x; 1.4200x over previous
import functools

import jax
import jax.numpy as jnp
from jax import lax
from jax.experimental import pallas as pl
from jax.experimental.pallas import tpu as pltpu
from jax.experimental.pallas import tpu_sc as plsc

N_NODES = 10000
N_EDGES = 320000
D = 128

NC = 2
NS = 16
NW = NC * NS
EPT = N_EDGES // NW
CH = 128
NCHUNK = -(-EPT // CH)
PAD_EPT = NCHUNK * CH
ACC_ROWS = 10240
DUMP_ROW = N_NODES
ZCH = ACC_ROWS // NS // CH
ROWS_PER_TILE_OUT = ACC_ROWS // NS

@functools.cache
def _mesh():
    return plsc.VectorSubcoreMesh(
        core_axis_name="c", subcore_axis_name="s",
        num_cores=NC, num_subcores=NS)


def _const_vmem_rows(ref, nrows, value):
    v16 = jnp.full((16,), value, jnp.float32)

    def body(i, _):
        for l in range(8):
            ref[i, pl.ds(l * 16, 16)] = v16
        return 0

    lax.fori_loop(0, nrows, body, 0)


def _sc_degree_body(dstp_hbm, degp_hbm, dst_v, ones_v, acc, sem):
    cid = lax.axis_index("c")
    sid = lax.axis_index("s")
    wid = sid * NC + cid
    pltpu.sync_copy(dstp_hbm.at[wid], dst_v)
    _const_vmem_rows(ones_v, CH, 0.0)

    def zero_chunk(i, _):
        pltpu.sync_copy(ones_v, acc.at[pl.ds(sid * (ZCH * CH) + i * CH, CH)])
        return 0

    lax.fori_loop(0, ZCH, zero_chunk, 0)
    _const_vmem_rows(ones_v, CH, 1.0)
    plsc.subcore_barrier()

    def edge_chunk(j, _):
        pltpu.sync_copy(ones_v, acc.at[dst_v.at[j]], add=True)
        return 0

    lax.fori_loop(0, NCHUNK, edge_chunk, 0)
    plsc.subcore_barrier()
    pltpu.sync_copy(
        acc.at[pl.ds(sid * ROWS_PER_TILE_OUT, ROWS_PER_TILE_OUT)],
        degp_hbm.at[cid, pl.ds(sid * ROWS_PER_TILE_OUT, ROWS_PER_TILE_OUT)],
    )


@functools.cache
def _sc_degree():
    return pl.kernel(
        _sc_degree_body,
        out_type=jax.ShapeDtypeStruct((NC, ACC_ROWS, D), jnp.float32),
        mesh=_mesh(),
        scratch_types=[
            pltpu.VMEM((NCHUNK, CH), jnp.int32),
            pltpu.VMEM((CH, D), jnp.float32),
            pltpu.VMEM_SHARED((ACC_ROWS, D), jnp.float32),
            pltpu.SemaphoreType.DMA,
        ],
    )


def _sc_aggregate_body(g_hbm, srcp_hbm, dstp_hbm, part_hbm,
                       src_v, dst_v, rows_v, acc, sem):
    cid = lax.axis_index("c")
    sid = lax.axis_index("s")
    wid = sid * NC + cid
    pltpu.sync_copy(srcp_hbm.at[wid], src_v)
    pltpu.sync_copy(dstp_hbm.at[wid], dst_v)
    _const_vmem_rows(rows_v, CH, 0.0)

    def zero_chunk(i, _):
        pltpu.sync_copy(rows_v, acc.at[pl.ds(sid * (ZCH * CH) + i * CH, CH)])
        return 0

    lax.fori_loop(0, ZCH, zero_chunk, 0)
    plsc.subcore_barrier()

    def edge_chunk(j, _):
        pltpu.async_copy(g_hbm.at[src_v.at[j]], rows_v, sem).wait()
        pltpu.sync_copy(rows_v, acc.at[dst_v.at[j]], add=True)
        return 0

    lax.fori_loop(0, NCHUNK, edge_chunk, 0)
    plsc.subcore_barrier()
    pltpu.sync_copy(
        acc.at[pl.ds(sid * ROWS_PER_TILE_OUT, ROWS_PER_TILE_OUT)],
        part_hbm.at[cid, pl.ds(sid * ROWS_PER_TILE_OUT, ROWS_PER_TILE_OUT)],
    )


@functools.cache
def _sc_aggregate():
    return pl.kernel(
        _sc_aggregate_body,
        out_type=jax.ShapeDtypeStruct((NC, ACC_ROWS, D), jnp.float32),
        mesh=_mesh(),
        scratch_types=[
            pltpu.VMEM((NCHUNK, CH), jnp.int32),
            pltpu.VMEM((NCHUNK, CH), jnp.int32),
            pltpu.VMEM((CH, D), jnp.float32),
            pltpu.VMEM_SHARED((ACC_ROWS, D), jnp.float32),
            pltpu.SemaphoreType.DMA,
        ],
    )


RB = 1000


def _dinv_from_degp(degp_blk):
    deg = 1.0 + degp_blk[0, :, 0:1] + degp_blk[1, :, 0:1]
    return lax.rsqrt(deg)


def _tc1_body(x_ref, w_ref, degp_ref, g_ref):
    dinv = _dinv_from_degp(degp_ref[...])
    h = jnp.dot(x_ref[...], w_ref[...], preferred_element_type=jnp.float32)
    g_ref[...] = h * dinv


def _tc2_body(p_ref, g_ref, degp_ref, w_ref, out_ref):
    dinv = _dinv_from_degp(degp_ref[...])
    s = (p_ref[0] + p_ref[1] + g_ref[...]) * dinv
    a = jnp.where(s >= 0, s, 0.3 * s)
    h = jnp.dot(a, w_ref[...], preferred_element_type=jnp.float32)
    out_ref[...] = h * dinv


def _tc3_body(p_ref, g_ref, degp_ref, out_ref):
    dinv = _dinv_from_degp(degp_ref[...])
    s = (p_ref[0] + p_ref[1] + g_ref[...]) * dinv
    out_ref[...] = jnp.maximum(s, 0.0)


_row_spec = pl.BlockSpec((RB, D), lambda i: (i, 0))
_part_spec = pl.BlockSpec((NC, RB, D), lambda i: (0, i, 0))
_degp_spec = pl.BlockSpec((NC, RB, D), lambda i: (0, i, 0))
_w_spec = pl.BlockSpec((D, D), lambda i: (0, 0))
_out_struct = jax.ShapeDtypeStruct((N_NODES, D), jnp.float32)

_tc1 = pl.pallas_call(
    _tc1_body, grid=(N_NODES // RB,),
    in_specs=[_row_spec, _w_spec, _degp_spec],
    out_specs=_row_spec, out_shape=_out_struct,
)
_tc2 = pl.pallas_call(
    _tc2_body, grid=(N_NODES // RB,),
    in_specs=[_part_spec, _row_spec, _degp_spec, _w_spec],
    out_specs=_row_spec, out_shape=_out_struct,
)
_tc3 = pl.pallas_call(
    _tc3_body, grid=(N_NODES // RB,),
    in_specs=[_part_spec, _row_spec, _degp_spec],
    out_specs=_row_spec, out_shape=_out_struct,
)


def kernel(x, edge_index, W1, W2):
    src = edge_index[0].astype(jnp.int32)
    dst = edge_index[1].astype(jnp.int32)
    pad = PAD_EPT - EPT
    srcp = jnp.pad(src.reshape(NW, EPT), ((0, 0), (0, pad))).reshape(
        NW, NCHUNK, CH)
    dump = DUMP_ROW + (jnp.arange(pad, dtype=jnp.int32)
                       % (ACC_ROWS - N_NODES))
    dstp = jnp.concatenate(
        [dst.reshape(NW, EPT), jnp.broadcast_to(dump, (NW, pad))],
        axis=1).reshape(NW, NCHUNK, CH)

    degp = _sc_degree()(dstp)
    g1 = _tc1(x, W1, degp)
    p1 = _sc_aggregate()(g1, srcp, dstp)
    g2 = _tc2(p1, g1, degp, W2)
    p2 = _sc_aggregate()(g2, srcp, dstp)
    return _tc3(p2, g2, degp)


# speedup vs baseline: 1.5133x; 1.0631x over previous
import functools

import jax
import jax.numpy as jnp
from jax import lax
from jax.experimental import pallas as pl
from jax.experimental.pallas import tpu as pltpu
from jax.experimental.pallas import tpu_sc as plsc

N_NODES = 10000
N_EDGES = 320000
D = 128

NC = 2
NS = 16
NW = NC * NS
EPT = N_EDGES // NW
CH = 128
NCHUNK = -(-EPT // CH)
PAD_EPT = NCHUNK * CH
ACC_ROWS = 10240
DUMP_ROW = N_NODES
ZCH = ACC_ROWS // NS // CH
ROWS_PER_TILE_OUT = ACC_ROWS // NS

@functools.cache
def _mesh():
    return plsc.VectorSubcoreMesh(
        core_axis_name="c", subcore_axis_name="s",
        num_cores=NC, num_subcores=NS)


def _const_vmem_rows(ref, nrows, value):
    v16 = jnp.full((16,), value, jnp.float32)

    def body(i, _):
        for l in range(8):
            ref[i, pl.ds(l * 16, 16)] = v16
        return 0

    lax.fori_loop(0, nrows, body, 0)


def _fill16_rows(ref, nrows, value):
    v16 = jnp.full((16,), value, jnp.float32)

    def body(i, _):
        ref[i, :] = v16
        return 0

    lax.fori_loop(0, nrows, body, 0)


def _sc_degree_body(dstp_hbm, degp_hbm, dst_v, ones_v, acc):
    cid = lax.axis_index("c")
    sid = lax.axis_index("s")
    wid = sid * NC + cid
    pltpu.sync_copy(dstp_hbm.at[wid], dst_v)
    _fill16_rows(ones_v, CH, 0.0)

    def zero_chunk(i, _):
        pltpu.sync_copy(ones_v, acc.at[pl.ds(sid * (ZCH * CH) + i * CH, CH)])
        return 0

    lax.fori_loop(0, ZCH, zero_chunk, 0)
    _fill16_rows(ones_v, CH, 1.0)
    plsc.subcore_barrier()

    def edge_chunk(j, _):
        pltpu.sync_copy(ones_v, acc.at[dst_v.at[j]], add=True)
        return 0

    lax.fori_loop(0, NCHUNK, edge_chunk, 0)
    plsc.subcore_barrier()
    pltpu.sync_copy(
        acc.at[pl.ds(sid * ROWS_PER_TILE_OUT, ROWS_PER_TILE_OUT)],
        degp_hbm.at[cid, pl.ds(sid * ROWS_PER_TILE_OUT, ROWS_PER_TILE_OUT)],
    )


@functools.cache
def _sc_degree():
    return pl.kernel(
        _sc_degree_body,
        out_type=jax.ShapeDtypeStruct((NC, ACC_ROWS, 16), jnp.float32),
        mesh=_mesh(),
        compiler_params=pltpu.CompilerParams(use_tc_tiling_on_sc=False),
        scratch_types=[
            pltpu.VMEM((NCHUNK, CH), jnp.int32),
            pltpu.VMEM((CH, 16), jnp.float32),
            pltpu.VMEM_SHARED((ACC_ROWS, 16), jnp.float32),
        ],
    )


def _sc_aggregate_body(g_hbm, srcp_hbm, dstp_hbm, part_hbm,
                       src_v, dst_v, rows_v, acc, sem):
    cid = lax.axis_index("c")
    sid = lax.axis_index("s")
    wid = sid * NC + cid
    pltpu.sync_copy(srcp_hbm.at[wid], src_v)
    pltpu.sync_copy(dstp_hbm.at[wid], dst_v)
    _const_vmem_rows(rows_v, CH, 0.0)

    def zero_chunk(i, _):
        pltpu.sync_copy(rows_v, acc.at[pl.ds(sid * (ZCH * CH) + i * CH, CH)])
        return 0

    lax.fori_loop(0, ZCH, zero_chunk, 0)
    plsc.subcore_barrier()

    def edge_chunk(j, _):
        pltpu.async_copy(g_hbm.at[src_v.at[j]], rows_v, sem).wait()
        pltpu.sync_copy(rows_v, acc.at[dst_v.at[j]], add=True)
        return 0

    lax.fori_loop(0, NCHUNK, edge_chunk, 0)
    plsc.subcore_barrier()
    pltpu.sync_copy(
        acc.at[pl.ds(sid * ROWS_PER_TILE_OUT, ROWS_PER_TILE_OUT)],
        part_hbm.at[cid, pl.ds(sid * ROWS_PER_TILE_OUT, ROWS_PER_TILE_OUT)],
    )


@functools.cache
def _sc_aggregate():
    return pl.kernel(
        _sc_aggregate_body,
        out_type=jax.ShapeDtypeStruct((NC, ACC_ROWS, D), jnp.float32),
        mesh=_mesh(),
        scratch_types=[
            pltpu.VMEM((NCHUNK, CH), jnp.int32),
            pltpu.VMEM((NCHUNK, CH), jnp.int32),
            pltpu.VMEM((CH, D), jnp.float32),
            pltpu.VMEM_SHARED((ACC_ROWS, D), jnp.float32),
            pltpu.SemaphoreType.DMA,
        ],
    )


RB = 1000


def _dinv_from_degp(degp_blk):
    deg = 1.0 + degp_blk[0, :, 0:1] + degp_blk[1, :, 0:1]
    return lax.rsqrt(deg)


def _tc1_body(x_ref, w_ref, degp_ref, g_ref):
    dinv = _dinv_from_degp(degp_ref[...])
    h = jnp.dot(x_ref[...], w_ref[...], preferred_element_type=jnp.float32)
    g_ref[...] = h * dinv


def _tc2_body(p_ref, g_ref, degp_ref, w_ref, out_ref):
    dinv = _dinv_from_degp(degp_ref[...])
    s = (p_ref[0] + p_ref[1] + g_ref[...]) * dinv
    a = jnp.where(s >= 0, s, 0.3 * s)
    h = jnp.dot(a, w_ref[...], preferred_element_type=jnp.float32)
    out_ref[...] = h * dinv


def _tc3_body(p_ref, g_ref, degp_ref, out_ref):
    dinv = _dinv_from_degp(degp_ref[...])
    s = (p_ref[0] + p_ref[1] + g_ref[...]) * dinv
    out_ref[...] = jnp.maximum(s, 0.0)


_row_spec = pl.BlockSpec((RB, D), lambda i: (i, 0))
_part_spec = pl.BlockSpec((NC, RB, D), lambda i: (0, i, 0))
_degp_spec = pl.BlockSpec((NC, RB, 16), lambda i: (0, i, 0))
_w_spec = pl.BlockSpec((D, D), lambda i: (0, 0))
_out_struct = jax.ShapeDtypeStruct((N_NODES, D), jnp.float32)

_tc1 = pl.pallas_call(
    _tc1_body, grid=(N_NODES // RB,),
    in_specs=[_row_spec, _w_spec, _degp_spec],
    out_specs=_row_spec, out_shape=_out_struct,
)
_tc2 = pl.pallas_call(
    _tc2_body, grid=(N_NODES // RB,),
    in_specs=[_part_spec, _row_spec, _degp_spec, _w_spec],
    out_specs=_row_spec, out_shape=_out_struct,
)
_tc3 = pl.pallas_call(
    _tc3_body, grid=(N_NODES // RB,),
    in_specs=[_part_spec, _row_spec, _degp_spec],
    out_specs=_row_spec, out_shape=_out_struct,
)


def kernel(x, edge_index, W1, W2):
    src = edge_index[0].astype(jnp.int32)
    dst = edge_index[1].astype(jnp.int32)
    pad = PAD_EPT - EPT
    srcp = jnp.pad(src.reshape(NW, EPT), ((0, 0), (0, pad))).reshape(
        NW, NCHUNK, CH)
    dump = DUMP_ROW + (jnp.arange(pad, dtype=jnp.int32)
                       % (ACC_ROWS - N_NODES))
    dstp = jnp.concatenate(
        [dst.reshape(NW, EPT), jnp.broadcast_to(dump, (NW, pad))],
        axis=1).reshape(NW, NCHUNK, CH)

    degp = _sc_degree()(dstp)
    g1 = _tc1(x, W1, degp)
    p1 = _sc_aggregate()(g1, srcp, dstp)
    g2 = _tc2(p1, g1, degp, W2)
    p2 = _sc_aggregate()(g2, srcp, dstp)
    return _tc3(p2, g2, degp)


# speedup vs baseline: 1.7382x; 1.1486x over previous
import functools

import jax
import jax.numpy as jnp
from jax import lax
from jax.experimental import pallas as pl
from jax.experimental.pallas import tpu as pltpu
from jax.experimental.pallas import tpu_sc as plsc

N_NODES = 10000
N_EDGES = 320000
D = 128

NC = 2
NS = 16
NW = NC * NS
EPT = N_EDGES // NW
CH = 128
NCHUNK = -(-EPT // CH)
PAD_EPT = NCHUNK * CH
ACC_ROWS = 10240
DUMP_ROW = N_NODES
ZCH = ACC_ROWS // NS // CH
ROWS_PER_TILE_OUT = ACC_ROWS // NS

@functools.cache
def _mesh():
    return plsc.VectorSubcoreMesh(
        core_axis_name="c", subcore_axis_name="s",
        num_cores=NC, num_subcores=NS)


def _const_vmem_rows(ref, nrows, value):
    v16 = jnp.full((16,), value, jnp.float32)

    def body(i, _):
        for l in range(8):
            ref[i, pl.ds(l * 16, 16)] = v16
        return 0

    lax.fori_loop(0, nrows, body, 0)


def _fill16_rows(ref, nrows, value):
    v16 = jnp.full((16,), value, jnp.float32)

    def body(i, _):
        ref[i, :] = v16
        return 0

    lax.fori_loop(0, nrows, body, 0)


def _sc_degree_body(dstp_hbm, degp_hbm, dst_v, ones_v, acc):
    cid = lax.axis_index("c")
    sid = lax.axis_index("s")
    wid = sid * NC + cid
    pltpu.sync_copy(dstp_hbm.at[wid], dst_v)
    _fill16_rows(ones_v, CH, 0.0)

    def zero_chunk(i, _):
        pltpu.sync_copy(ones_v, acc.at[pl.ds(sid * (ZCH * CH) + i * CH, CH)])
        return 0

    lax.fori_loop(0, ZCH, zero_chunk, 0)
    _fill16_rows(ones_v, CH, 1.0)
    plsc.subcore_barrier()

    def edge_chunk(j, _):
        pltpu.sync_copy(ones_v, acc.at[dst_v.at[j]], add=True)
        return 0

    lax.fori_loop(0, NCHUNK, edge_chunk, 0)
    plsc.subcore_barrier()
    pltpu.sync_copy(
        acc.at[pl.ds(sid * ROWS_PER_TILE_OUT, ROWS_PER_TILE_OUT)],
        degp_hbm.at[cid, pl.ds(sid * ROWS_PER_TILE_OUT, ROWS_PER_TILE_OUT)],
    )


@functools.cache
def _sc_degree():
    return pl.kernel(
        _sc_degree_body,
        out_type=jax.ShapeDtypeStruct((NC, ACC_ROWS, 16), jnp.float32),
        mesh=_mesh(),
        compiler_params=pltpu.CompilerParams(use_tc_tiling_on_sc=False),
        scratch_types=[
            pltpu.VMEM((NCHUNK, CH), jnp.int32),
            pltpu.VMEM((CH, 16), jnp.float32),
            pltpu.VMEM_SHARED((ACC_ROWS, 16), jnp.float32),
        ],
    )


def _sc_aggregate_body(g_hbm, pk_hbm, part_hbm,
                       pk_v, sring, dring, rows_a, rows_b, acc,
                       ga, gb, sa, sb):
    cid = lax.axis_index("c")
    sid = lax.axis_index("s")
    wid = sid * NC + cid
    pltpu.sync_copy(pk_hbm.at[wid], pk_v)
    _const_vmem_rows(rows_a, CH, 0.0)

    def zero_chunk(i, _):
        pltpu.sync_copy(rows_a, acc.at[pl.ds(sid * (ZCH * CH) + i * CH, CH)])
        return 0

    lax.fori_loop(0, ZCH, zero_chunk, 0)
    plsc.subcore_barrier()

    def unpack(j, r):
        for i in range(8):
            v = pk_v[j, pl.ds(i * 16, 16)]
            sring[r, pl.ds(i * 16, 16)] = v & 0xFFFF
            dring[r, pl.ds(i * 16, 16)] = v >> 16

    unpack(0, 0)
    pltpu.async_copy(g_hbm.at[sring.at[0]], rows_a, ga)

    def pair(j2, _):
        c0 = j2 * 2
        @pl.when(j2 > 0)
        def _():
            pltpu.make_async_copy(rows_b, acc.at[dring.at[1]], sb).wait()
        unpack(c0 + 1, 1)
        pltpu.make_async_copy(g_hbm.at[sring.at[0]], rows_a, ga).wait()
        pltpu.async_copy(g_hbm.at[sring.at[1]], rows_b, gb)
        pltpu.async_copy(rows_a, acc.at[dring.at[0]], sa, add=True)
        pltpu.make_async_copy(rows_a, acc.at[dring.at[0]], sa).wait()
        unpack(c0 + 2, 0)
        pltpu.make_async_copy(g_hbm.at[sring.at[1]], rows_b, gb).wait()
        pltpu.async_copy(g_hbm.at[sring.at[0]], rows_a, ga)
        pltpu.async_copy(rows_b, acc.at[dring.at[1]], sb, add=True)
        return 0

    lax.fori_loop(0, (NCHUNK - 1) // 2, pair, 0)
    pltpu.make_async_copy(rows_b, acc.at[dring.at[1]], sb).wait()
    pltpu.make_async_copy(g_hbm.at[sring.at[0]], rows_a, ga).wait()
    pltpu.sync_copy(rows_a, acc.at[dring.at[0]], add=True)
    plsc.subcore_barrier()
    pltpu.sync_copy(
        acc.at[pl.ds(sid * ROWS_PER_TILE_OUT, ROWS_PER_TILE_OUT)],
        part_hbm.at[cid, pl.ds(sid * ROWS_PER_TILE_OUT, ROWS_PER_TILE_OUT)],
    )


@functools.cache
def _sc_aggregate():
    return pl.kernel(
        _sc_aggregate_body,
        out_type=jax.ShapeDtypeStruct((NC, ACC_ROWS, D), jnp.float32),
        mesh=_mesh(),
        scratch_types=[
            pltpu.VMEM((NCHUNK, CH), jnp.int32),
            pltpu.VMEM((2, CH), jnp.int32),
            pltpu.VMEM((2, CH), jnp.int32),
            pltpu.VMEM((CH, D), jnp.float32),
            pltpu.VMEM((CH, D), jnp.float32),
            pltpu.VMEM_SHARED((ACC_ROWS, D), jnp.float32),
            pltpu.SemaphoreType.DMA,
            pltpu.SemaphoreType.DMA,
            pltpu.SemaphoreType.DMA,
            pltpu.SemaphoreType.DMA,
        ],
    )


RB = 1000


def _dinv_from_degp(degp_blk):
    deg = 1.0 + degp_blk[0, :, 0:1] + degp_blk[1, :, 0:1]
    return lax.rsqrt(deg)


def _tc1_body(x_ref, w_ref, degp_ref, g_ref):
    dinv = _dinv_from_degp(degp_ref[...])
    h = jnp.dot(x_ref[...], w_ref[...], preferred_element_type=jnp.float32)
    g_ref[...] = h * dinv


def _tc2_body(p_ref, g_ref, degp_ref, w_ref, out_ref):
    dinv = _dinv_from_degp(degp_ref[...])
    s = (p_ref[0] + p_ref[1] + g_ref[...]) * dinv
    a = jnp.where(s >= 0, s, 0.3 * s)
    h = jnp.dot(a, w_ref[...], preferred_element_type=jnp.float32)
    out_ref[...] = h * dinv


def _tc3_body(p_ref, g_ref, degp_ref, out_ref):
    dinv = _dinv_from_degp(degp_ref[...])
    s = (p_ref[0] + p_ref[1] + g_ref[...]) * dinv
    out_ref[...] = jnp.maximum(s, 0.0)


_row_spec = pl.BlockSpec((RB, D), lambda i: (i, 0))
_part_spec = pl.BlockSpec((NC, RB, D), lambda i: (0, i, 0))
_degp_spec = pl.BlockSpec((NC, RB, 16), lambda i: (0, i, 0))
_w_spec = pl.BlockSpec((D, D), lambda i: (0, 0))
_out_struct = jax.ShapeDtypeStruct((N_NODES, D), jnp.float32)

_tc1 = pl.pallas_call(
    _tc1_body, grid=(N_NODES // RB,),
    in_specs=[_row_spec, _w_spec, _degp_spec],
    out_specs=_row_spec, out_shape=_out_struct,
)
_tc2 = pl.pallas_call(
    _tc2_body, grid=(N_NODES // RB,),
    in_specs=[_part_spec, _row_spec, _degp_spec, _w_spec],
    out_specs=_row_spec, out_shape=_out_struct,
)
_tc3 = pl.pallas_call(
    _tc3_body, grid=(N_NODES // RB,),
    in_specs=[_part_spec, _row_spec, _degp_spec],
    out_specs=_row_spec, out_shape=_out_struct,
)


def kernel(x, edge_index, W1, W2):
    src = edge_index[0].astype(jnp.int32)
    dst = edge_index[1].astype(jnp.int32)
    pad = PAD_EPT - EPT
    srcp = jnp.pad(src.reshape(NW, EPT), ((0, 0), (0, pad))).reshape(
        NW, NCHUNK, CH)
    dump = DUMP_ROW + (jnp.arange(pad, dtype=jnp.int32)
                       % (ACC_ROWS - N_NODES))
    dstp = jnp.concatenate(
        [dst.reshape(NW, EPT), jnp.broadcast_to(dump, (NW, pad))],
        axis=1).reshape(NW, NCHUNK, CH)
    pk = srcp | (dstp << 16)

    degp = _sc_degree()(dstp)
    g1 = _tc1(x, W1, degp)
    p1 = _sc_aggregate()(g1, pk)
    g2 = _tc2(p1, g1, degp, W2)
    p2 = _sc_aggregate()(g2, pk)
    return _tc3(p2, g2, degp)


# speedup vs baseline: 1.7465x; 1.0048x over previous
import functools

import jax
import jax.numpy as jnp
from jax import lax
from jax.experimental import pallas as pl
from jax.experimental.pallas import tpu as pltpu
from jax.experimental.pallas import tpu_sc as plsc

N_NODES = 10000
N_EDGES = 320000
D = 128

NC = 2
NS = 16
NW = NC * NS
EPT = N_EDGES // NW
CH = 128
NCHUNK = -(-EPT // CH)
PAD_EPT = NCHUNK * CH
ACC_ROWS = 10240
DUMP_ROW = N_NODES
ZCH = ACC_ROWS // NS // CH
ROWS_PER_TILE_OUT = ACC_ROWS // NS

@functools.cache
def _mesh():
    return plsc.VectorSubcoreMesh(
        core_axis_name="c", subcore_axis_name="s",
        num_cores=NC, num_subcores=NS)


def _const_vmem_rows(ref, nrows, value):
    v16 = jnp.full((16,), value, jnp.float32)

    def body(i, _):
        for l in range(8):
            ref[i, pl.ds(l * 16, 16)] = v16
        return 0

    lax.fori_loop(0, nrows, body, 0)


def _fill16_rows(ref, nrows, value):
    v16 = jnp.full((16,), value, jnp.float32)

    def body(i, _):
        ref[i, :] = v16
        return 0

    lax.fori_loop(0, nrows, body, 0)


def _sc_degree_body(dstp_hbm, degp_hbm, dst_v, ones_v, acc, sem):
    cid = lax.axis_index("c")
    sid = lax.axis_index("s")
    wid = sid * NC + cid
    pltpu.sync_copy(dstp_hbm.at[wid], dst_v)
    _fill16_rows(ones_v, CH, 0.0)

    def zero_chunk(i, _):
        pltpu.sync_copy(ones_v, acc.at[pl.ds(sid * (ZCH * CH) + i * CH, CH)])
        return 0

    lax.fori_loop(0, ZCH, zero_chunk, 0)
    _fill16_rows(ones_v, CH, 1.0)
    plsc.subcore_barrier()

    def edge_chunk(j, _):
        pltpu.async_copy(ones_v, acc.at[dst_v.at[j]], sem, add=True)
        return 0

    lax.fori_loop(0, NCHUNK, edge_chunk, 0)

    def drain(j, _):
        pltpu.make_async_copy(ones_v, acc.at[dst_v.at[j]], sem).wait()
        return 0

    lax.fori_loop(0, NCHUNK, drain, 0)
    plsc.subcore_barrier()
    pltpu.sync_copy(
        acc.at[pl.ds(sid * ROWS_PER_TILE_OUT, ROWS_PER_TILE_OUT)],
        degp_hbm.at[cid, pl.ds(sid * ROWS_PER_TILE_OUT, ROWS_PER_TILE_OUT)],
    )


@functools.cache
def _sc_degree():
    return pl.kernel(
        _sc_degree_body,
        out_type=jax.ShapeDtypeStruct((NC, ACC_ROWS, 16), jnp.float32),
        mesh=_mesh(),
        compiler_params=pltpu.CompilerParams(use_tc_tiling_on_sc=False),
        scratch_types=[
            pltpu.VMEM((NCHUNK, CH), jnp.int32),
            pltpu.VMEM((CH, 16), jnp.float32),
            pltpu.VMEM_SHARED((ACC_ROWS, 16), jnp.float32),
            pltpu.SemaphoreType.DMA,
        ],
    )


def _sc_aggregate_body(g_hbm, pk_hbm, part_hbm,
                       pk_v, sring, dring, rows_a, rows_b, acc,
                       ga, gb, sa, sb):
    cid = lax.axis_index("c")
    sid = lax.axis_index("s")
    wid = sid * NC + cid
    pltpu.sync_copy(pk_hbm.at[wid], pk_v)
    _const_vmem_rows(rows_a, CH, 0.0)

    def zero_chunk(i, _):
        pltpu.sync_copy(rows_a, acc.at[pl.ds(sid * (ZCH * CH) + i * CH, CH)])
        return 0

    lax.fori_loop(0, ZCH, zero_chunk, 0)
    plsc.subcore_barrier()

    def unpack(j, r):
        for i in range(8):
            v = pk_v[j, pl.ds(i * 16, 16)]
            sring[r, pl.ds(i * 16, 16)] = v & 0xFFFF
            dring[r, pl.ds(i * 16, 16)] = v >> 16

    unpack(0, 0)
    pltpu.async_copy(g_hbm.at[sring.at[0]], rows_a, ga)

    def pair(j2, _):
        c0 = j2 * 2
        @pl.when(j2 > 0)
        def _():
            pltpu.make_async_copy(rows_b, acc.at[dring.at[1]], sb).wait()
        unpack(c0 + 1, 1)
        pltpu.make_async_copy(g_hbm.at[sring.at[0]], rows_a, ga).wait()
        pltpu.async_copy(g_hbm.at[sring.at[1]], rows_b, gb)
        pltpu.async_copy(rows_a, acc.at[dring.at[0]], sa, add=True)
        pltpu.make_async_copy(rows_a, acc.at[dring.at[0]], sa).wait()
        unpack(c0 + 2, 0)
        pltpu.make_async_copy(g_hbm.at[sring.at[1]], rows_b, gb).wait()
        pltpu.async_copy(g_hbm.at[sring.at[0]], rows_a, ga)
        pltpu.async_copy(rows_b, acc.at[dring.at[1]], sb, add=True)
        return 0

    lax.fori_loop(0, (NCHUNK - 1) // 2, pair, 0)
    pltpu.make_async_copy(rows_b, acc.at[dring.at[1]], sb).wait()
    pltpu.make_async_copy(g_hbm.at[sring.at[0]], rows_a, ga).wait()
    pltpu.sync_copy(rows_a, acc.at[dring.at[0]], add=True)
    plsc.subcore_barrier()
    pltpu.sync_copy(
        acc.at[pl.ds(sid * ROWS_PER_TILE_OUT, ROWS_PER_TILE_OUT)],
        part_hbm.at[cid, pl.ds(sid * ROWS_PER_TILE_OUT, ROWS_PER_TILE_OUT)],
    )


@functools.cache
def _sc_aggregate():
    return pl.kernel(
        _sc_aggregate_body,
        out_type=jax.ShapeDtypeStruct((NC, ACC_ROWS, D), jnp.float32),
        mesh=_mesh(),
        scratch_types=[
            pltpu.VMEM((NCHUNK, CH), jnp.int32),
            pltpu.VMEM((2, CH), jnp.int32),
            pltpu.VMEM((2, CH), jnp.int32),
            pltpu.VMEM((CH, D), jnp.float32),
            pltpu.VMEM((CH, D), jnp.float32),
            pltpu.VMEM_SHARED((ACC_ROWS, D), jnp.float32),
            pltpu.SemaphoreType.DMA,
            pltpu.SemaphoreType.DMA,
            pltpu.SemaphoreType.DMA,
            pltpu.SemaphoreType.DMA,
        ],
    )


RB = 1000


def _dinv_from_degp(degp_blk):
    deg = 1.0 + degp_blk[0, :, 0:1] + degp_blk[1, :, 0:1]
    return lax.rsqrt(deg)


def _tc1_body(x_ref, w_ref, degp_ref, g_ref):
    dinv = _dinv_from_degp(degp_ref[...])
    h = jnp.dot(x_ref[...], w_ref[...], preferred_element_type=jnp.float32)
    g_ref[...] = h * dinv


def _tc2_body(p_ref, g_ref, degp_ref, w_ref, out_ref):
    dinv = _dinv_from_degp(degp_ref[...])
    s = (p_ref[0] + p_ref[1] + g_ref[...]) * dinv
    a = jnp.where(s >= 0, s, 0.3 * s)
    h = jnp.dot(a, w_ref[...], preferred_element_type=jnp.float32)
    out_ref[...] = h * dinv


def _tc3_body(p_ref, g_ref, degp_ref, out_ref):
    dinv = _dinv_from_degp(degp_ref[...])
    s = (p_ref[0] + p_ref[1] + g_ref[...]) * dinv
    out_ref[...] = jnp.maximum(s, 0.0)


_row_spec = pl.BlockSpec((RB, D), lambda i: (i, 0))
_part_spec = pl.BlockSpec((NC, RB, D), lambda i: (0, i, 0))
_degp_spec = pl.BlockSpec((NC, RB, 16), lambda i: (0, i, 0))
_w_spec = pl.BlockSpec((D, D), lambda i: (0, 0))
_out_struct = jax.ShapeDtypeStruct((N_NODES, D), jnp.float32)

_tc1 = pl.pallas_call(
    _tc1_body, grid=(N_NODES // RB,),
    in_specs=[_row_spec, _w_spec, _degp_spec],
    out_specs=_row_spec, out_shape=_out_struct,
)
_tc2 = pl.pallas_call(
    _tc2_body, grid=(N_NODES // RB,),
    in_specs=[_part_spec, _row_spec, _degp_spec, _w_spec],
    out_specs=_row_spec, out_shape=_out_struct,
)
_tc3 = pl.pallas_call(
    _tc3_body, grid=(N_NODES // RB,),
    in_specs=[_part_spec, _row_spec, _degp_spec],
    out_specs=_row_spec, out_shape=_out_struct,
)


def kernel(x, edge_index, W1, W2):
    src = edge_index[0].astype(jnp.int32)
    dst = edge_index[1].astype(jnp.int32)
    pad = PAD_EPT - EPT
    srcp = jnp.pad(src.reshape(NW, EPT), ((0, 0), (0, pad))).reshape(
        NW, NCHUNK, CH)
    dump = DUMP_ROW + (jnp.arange(pad, dtype=jnp.int32)
                       % (ACC_ROWS - N_NODES))
    dstp = jnp.concatenate(
        [dst.reshape(NW, EPT), jnp.broadcast_to(dump, (NW, pad))],
        axis=1).reshape(NW, NCHUNK, CH)
    pk = srcp | (dstp << 16)

    degp = _sc_degree()(dstp)
    g1 = _tc1(x, W1, degp)
    p1 = _sc_aggregate()(g1, pk)
    g2 = _tc2(p1, g1, degp, W2)
    p2 = _sc_aggregate()(g2, pk)
    return _tc3(p2, g2, degp)
